# Initial kernel scaffold; baseline (speedup 1.0000x reference)
import jax
import jax.numpy as jnp
from jax import lax
from jax.experimental import pallas as pl
from jax.experimental.pallas import tpu as pltpu
from jax.experimental.pallas import tpu_sc as plsc

NF = 26
V = 100000
D = 32
B = 16384
NUM = 13
IN_DIM = NF * D + NUM

NC = 2
NS = 16
NW = NC * NS
P = B * NF // NW
PR = P // 128
GS = 8
GROUP = GS * 128
G = P // GROUP


def _sc_gather_body(cat_hbm, tab_hbm, out_hbm, idx_v, buf0, buf1, gsem, osem):
    wid = lax.axis_index("s") * NC + lax.axis_index("c")
    row0 = wid * PR
    look0 = wid * P

    pltpu.sync_copy(cat_hbm.at[pl.ds(row0, PR)], idx_v)

    iota = lax.broadcasted_iota(jnp.int32, (16,), 0)

    def fix_row(r, carry):
        for j in range(8):
            q = r * 128 + j * 16 + iota
            f = lax.rem(q, jnp.int32(NF))
            cur = idx_v[r, pl.ds(j * 16, 16)]
            idx_v[r, pl.ds(j * 16, 16)] = cur + f * jnp.int32(V)
        return carry

    lax.fori_loop(0, PR, fix_row, 0)

    out_handles = [None] * G
    for g in range(G):
        buf = buf0 if g % 2 == 0 else buf1
        if g >= 2:
            out_handles[g - 2].wait()
        ghs = []
        for j in range(GS):
            ghs.append(
                pltpu.async_copy(
                    tab_hbm.at[idx_v.at[g * GS + j]],
                    buf.at[pl.ds(j * 128, 128)],
                    gsem,
                )
            )
        for h in ghs:
            h.wait()
        out_handles[g] = pltpu.async_copy(
            buf, out_hbm.at[pl.ds(look0 + g * GROUP, GROUP)], osem
        )
    out_handles[G - 2].wait()
    out_handles[G - 1].wait()


_sc_gather = pl.kernel(
    _sc_gather_body,
    out_type=jax.ShapeDtypeStruct((B * NF, D), jnp.float32),
    mesh=plsc.VectorSubcoreMesh(
        core_axis_name="c", subcore_axis_name="s", num_cores=NC, num_subcores=NS
    ),
    scratch_types=[
        pltpu.VMEM((PR, 128), jnp.int32),
        pltpu.VMEM((GROUP, D), jnp.float32),
        pltpu.VMEM((GROUP, D), jnp.float32),
        pltpu.SemaphoreType.DMA,
        pltpu.SemaphoreType.DMA,
    ],
)


R = 1024


def _mlp_body(emb_ref, num_ref, w1_ref, b1_ref, w2_ref, b2_ref, w3_ref,
              b3_ref, w4_ref, b4_ref, out_ref):
    x = emb_ref[...]
    n = num_ref[...]
    w1a = w1_ref[0:NF * D, :]
    w1b = w1_ref[NF * D:IN_DIM, :]
    h = (jnp.dot(x, w1a, preferred_element_type=jnp.float32)
         + jnp.dot(n, w1b, preferred_element_type=jnp.float32)
         + b1_ref[...][None, :])
    h = jnp.maximum(h, 0.0)
    h = jnp.dot(h, w2_ref[...], preferred_element_type=jnp.float32) + b2_ref[...][None, :]
    h = jnp.maximum(h, 0.0)
    h = jnp.dot(h, w3_ref[...], preferred_element_type=jnp.float32) + b3_ref[...][None, :]
    h = jnp.maximum(h, 0.0)
    o = jnp.dot(h, w4_ref[...], preferred_element_type=jnp.float32) + b4_ref[...][None, :]
    out_ref[...] = o


def _mlp(emb, num_data, W1, b1, W2, b2, W3, b3, W4, b4):
    grid = (B // R,)
    full = lambda i: (0, 0)
    return pl.pallas_call(
        _mlp_body,
        grid=grid,
        in_specs=[
            pl.BlockSpec((R, NF * D), lambda i: (i, 0)),
            pl.BlockSpec((R, NUM), lambda i: (i, 0)),
            pl.BlockSpec((IN_DIM, 128), full),
            pl.BlockSpec((128,), lambda i: (0,)),
            pl.BlockSpec((128, 64), full),
            pl.BlockSpec((64,), lambda i: (0,)),
            pl.BlockSpec((64, 32), full),
            pl.BlockSpec((32,), lambda i: (0,)),
            pl.BlockSpec((32, 1), full),
            pl.BlockSpec((1,), lambda i: (0,)),
        ],
        out_specs=pl.BlockSpec((R, 1), lambda i: (i, 0)),
        out_shape=jax.ShapeDtypeStruct((B, 1), jnp.float32),
    )(emb, num_data, W1, b1, W2, b2, W3, b3, W4, b4)


def kernel(cat_data, num_data, tables, W1, b1, W2, b2, W3, b3, W4, b4):
    cat2d = cat_data.reshape(B * NF // 128, 128)
    tab_flat = tables.reshape(NF * V, D)
    emb_flat = _sc_gather(cat2d, tab_flat)
    emb = emb_flat.reshape(B, NF * D)
    out = _mlp(emb, num_data, W1, b1, W2, b2, W3, b3, W4, b4)
    return out[:, 0]


# speedup vs baseline: 8.0804x; 8.0804x over previous
import jax
import jax.numpy as jnp
from jax import lax
from jax.experimental import pallas as pl
from jax.experimental.pallas import tpu as pltpu
from jax.experimental.pallas import tpu_sc as plsc

NF = 26
V = 100000
D = 32
B = 16384
NUM = 13
IN_DIM = NF * D + NUM

NC = 2
NS = 16
NW = NC * NS
P = B * NF // NW
PR = P // 128
GS = 8
GROUP = GS * 128
G = P // GROUP


def _sc_gather_body(cat_hbm, tab_hbm, out_hbm, idx_v, buf0, buf1, gsem, osem):
    wid = lax.axis_index("s") * NC + lax.axis_index("c")
    row0 = wid * PR
    look0 = wid * P

    pltpu.sync_copy(cat_hbm.at[pl.ds(row0, PR)], idx_v)

    iota = lax.broadcasted_iota(jnp.int32, (16,), 0)

    def fix_row(r, carry):
        for j in range(8):
            q = r * 128 + j * 16 + iota
            f = lax.rem(q, jnp.int32(NF))
            cur = idx_v[r, pl.ds(j * 16, 16)]
            idx_v[r, pl.ds(j * 16, 16)] = cur + f * jnp.int32(V)
        return carry

    lax.fori_loop(0, PR, fix_row, 0)

    out_handles = [None] * G
    for g in range(G):
        buf = buf0 if g % 2 == 0 else buf1
        if g >= 2:
            out_handles[g - 2].wait()
        ghs = []
        for j in range(GS):
            ghs.append(
                pltpu.async_copy(
                    tab_hbm.at[idx_v.at[g * GS + j]],
                    buf.at[pl.ds(j * 128, 128)],
                    gsem,
                )
            )
        for h in ghs:
            h.wait()
        out_handles[g] = pltpu.async_copy(
            buf, out_hbm.at[pl.ds(look0 + g * GROUP, GROUP)], osem
        )
    out_handles[G - 2].wait()
    out_handles[G - 1].wait()


def _make_sc_gather():
    return pl.kernel(
        _sc_gather_body,
        out_type=jax.ShapeDtypeStruct((B * NF, D), jnp.float32),
        mesh=plsc.VectorSubcoreMesh(
            core_axis_name="c", subcore_axis_name="s",
            num_cores=NC, num_subcores=NS,
        ),
        scratch_types=[
            pltpu.VMEM((PR, 128), jnp.int32),
            pltpu.VMEM((GROUP, D), jnp.float32),
            pltpu.VMEM((GROUP, D), jnp.float32),
            pltpu.SemaphoreType.DMA,
            pltpu.SemaphoreType.DMA,
        ],
        compiler_params=pltpu.CompilerParams(use_tc_tiling_on_sc=False),
    )


R = 1024


def _mlp_body(emb_ref, num_ref, w1_ref, b1_ref, w2_ref, b2_ref, w3_ref,
              b3_ref, w4_ref, b4_ref, out_ref):
    x = emb_ref[...]
    n = num_ref[...]
    w1a = w1_ref[0:NF * D, :]
    w1b = w1_ref[NF * D:IN_DIM, :]
    h = (jnp.dot(x, w1a, preferred_element_type=jnp.float32)
         + jnp.dot(n, w1b, preferred_element_type=jnp.float32)
         + b1_ref[...][None, :])
    h = jnp.maximum(h, 0.0)
    h = jnp.dot(h, w2_ref[...], preferred_element_type=jnp.float32) + b2_ref[...][None, :]
    h = jnp.maximum(h, 0.0)
    h = jnp.dot(h, w3_ref[...], preferred_element_type=jnp.float32) + b3_ref[...][None, :]
    h = jnp.maximum(h, 0.0)
    o = jnp.dot(h, w4_ref[...], preferred_element_type=jnp.float32) + b4_ref[...][None, :]
    out_ref[...] = o


def _mlp(emb, num_data, W1, b1, W2, b2, W3, b3, W4, b4):
    grid = (B // R,)
    full = lambda i: (0, 0)
    return pl.pallas_call(
        _mlp_body,
        grid=grid,
        in_specs=[
            pl.BlockSpec((R, NF * D), lambda i: (i, 0)),
            pl.BlockSpec((R, NUM), lambda i: (i, 0)),
            pl.BlockSpec((IN_DIM, 128), full),
            pl.BlockSpec((128,), lambda i: (0,)),
            pl.BlockSpec((128, 64), full),
            pl.BlockSpec((64,), lambda i: (0,)),
            pl.BlockSpec((64, 32), full),
            pl.BlockSpec((32,), lambda i: (0,)),
            pl.BlockSpec((32, 1), full),
            pl.BlockSpec((1,), lambda i: (0,)),
        ],
        out_specs=pl.BlockSpec((R, 1), lambda i: (i, 0)),
        out_shape=jax.ShapeDtypeStruct((B, 1), jnp.float32),
    )(emb, num_data, W1, b1, W2, b2, W3, b3, W4, b4)


def kernel(cat_data, num_data, tables, W1, b1, W2, b2, W3, b3, W4, b4):
    cat2d = cat_data.reshape(B * NF // 128, 128)
    tab_flat = tables.reshape(NF * V, D)
    emb_flat = _make_sc_gather()(cat2d, tab_flat)
    emb = emb_flat.reshape(B, NF * D)
    out = _mlp(emb, num_data, W1, b1, W2, b2, W3, b3, W4, b4)
    return out[:, 0]
